# Initial kernel scaffold; baseline (speedup 1.0000x reference)
import jax
import jax.numpy as jnp
from jax.experimental import pallas as pl


def kernel(features, adj, weight0, weight1, bias):
    raise NotImplementedError("write your pallas kernel here")


# speedup vs baseline: 1.0675x; 1.0675x over previous
import jax
import jax.numpy as jnp
from jax.experimental import pallas as pl
from jax.experimental.pallas import tpu as pltpu


def _conv_body(BM, x_ref, w0_ref, w1_ref, b_ref, a_ref, out_ref, s1_ref):
    i = pl.program_id(0)

    @pl.when(i == 0)
    def _init_s1():
        xb = x_ref[...].astype(jnp.bfloat16)
        w1 = w1_ref[...].astype(jnp.bfloat16)
        s1_ref[...] = jnp.dot(
            xb, w1, preferred_element_type=jnp.float32
        ).astype(jnp.bfloat16)

    agg = jnp.dot(
        a_ref[...].astype(jnp.bfloat16),
        s1_ref[...],
        preferred_element_type=jnp.float32,
    )
    x_rows = x_ref[pl.ds(i * BM, BM), :].astype(jnp.bfloat16)
    s0 = jnp.dot(
        x_rows, w0_ref[...].astype(jnp.bfloat16),
        preferred_element_type=jnp.float32,
    )
    out_ref[...] = s0 + agg + b_ref[...]


def kernel(features, adj, weight0, weight1, bias):
    n, d_in = features.shape
    d_out = weight0.shape[1]

    BM = 200
    assert n % BM == 0, (n, BM)
    grid = (n // BM,)

    bias2d = bias.reshape(1, d_out)

    body = lambda *refs: _conv_body(BM, *refs)

    out = pl.pallas_call(
        body,
        grid=grid,
        in_specs=[
            pl.BlockSpec((n, d_in), lambda i: (0, 0)),
            pl.BlockSpec((d_in, d_out), lambda i: (0, 0)),
            pl.BlockSpec((d_in, d_out), lambda i: (0, 0)),
            pl.BlockSpec((1, d_out), lambda i: (0, 0)),
            pl.BlockSpec((BM, n), lambda i: (i, 0)),
        ],
        out_specs=pl.BlockSpec((BM, d_out), lambda i: (i, 0)),
        out_shape=jax.ShapeDtypeStruct((n, d_out), jnp.float32),
        scratch_shapes=[pltpu.VMEM((n, d_out), jnp.bfloat16)],
    )(features, weight0, weight1, bias2d, adj)
    return out


# speedup vs baseline: 1.0833x; 1.0148x over previous
import jax
import jax.numpy as jnp
from jax.experimental import pallas as pl
from jax.experimental.pallas import tpu as pltpu


def _conv_body(BM, x_ref, w0_ref, w1_ref, b_ref, a_ref, out_ref, s1_ref):
    i = pl.program_id(0)

    @pl.when(i == 0)
    def _init_s1():
        xb = x_ref[...].astype(jnp.bfloat16)
        w1 = w1_ref[...].astype(jnp.bfloat16)
        s1_ref[...] = jnp.dot(
            xb, w1, preferred_element_type=jnp.float32
        ).astype(jnp.bfloat16)

    agg = jnp.dot(
        a_ref[...].astype(jnp.bfloat16),
        s1_ref[...],
        preferred_element_type=jnp.float32,
    )
    x_rows = x_ref[pl.ds(i * BM, BM), :].astype(jnp.bfloat16)
    s0 = jnp.dot(
        x_rows, w0_ref[...].astype(jnp.bfloat16),
        preferred_element_type=jnp.float32,
    )
    out_ref[...] = s0 + agg + b_ref[...]


def kernel(features, adj, weight0, weight1, bias):
    n, d_in = features.shape
    d_out = weight0.shape[1]

    BM = 400
    assert n % BM == 0, (n, BM)
    grid = (n // BM,)

    bias2d = bias.reshape(1, d_out)

    body = lambda *refs: _conv_body(BM, *refs)

    out = pl.pallas_call(
        body,
        grid=grid,
        in_specs=[
            pl.BlockSpec((n, d_in), lambda i: (0, 0)),
            pl.BlockSpec((d_in, d_out), lambda i: (0, 0)),
            pl.BlockSpec((d_in, d_out), lambda i: (0, 0)),
            pl.BlockSpec((1, d_out), lambda i: (0, 0)),
            pl.BlockSpec((BM, n), lambda i: (i, 0)),
        ],
        out_specs=pl.BlockSpec((BM, d_out), lambda i: (i, 0)),
        out_shape=jax.ShapeDtypeStruct((n, d_out), jnp.float32),
        scratch_shapes=[pltpu.VMEM((n, d_out), jnp.bfloat16)],
    )(features, weight0, weight1, bias2d, adj)
    return out
